# baseline (speedup 1.0000x reference)
import jax
import jax.numpy as jnp
from jax.experimental import pallas as pl

GAMMA_K = 0.1
_INT_MIN = -(2 ** 31)


def _select_mask_body(x_ref, o_ref, *, kth: int):
    xb = x_ref[0]
    xz = xb + 0.0
    b = jax.lax.bitcast_convert_type(xz, jnp.int32)
    u = jnp.where(b < 0, b ^ jnp.int32(0x7FFFFFFF), b)
    imin = jnp.int32(_INT_MIN)

    def it(i, zb):
        bit = jnp.int32(31) - i
        cand = zb | jnp.left_shift(jnp.int32(1), bit)
        z = cand ^ imin
        part = jnp.sum((u >= z).astype(jnp.int32), axis=0)
        cnt = jnp.sum(part)
        return jnp.where(cnt >= kth, cand, zb)

    zb = jax.lax.fori_loop(0, 32, it, jnp.int32(0))
    t = zb ^ imin
    o_ref[0] = jnp.where(u < t, xb, 0.0)


def kernel(x):
    B, C, H, W = x.shape
    n = C * H * W
    kth = int(GAMMA_K * n)
    lanes = 1024
    slabs = n // (8 * lanes)
    xf = x.reshape(B, slabs, 8, lanes)

    out = pl.pallas_call(
        lambda x_ref, o_ref: _select_mask_body(x_ref, o_ref, kth=kth),
        grid=(B,),
        in_specs=[pl.BlockSpec((1, slabs, 8, lanes), lambda i: (i, 0, 0, 0))],
        out_specs=pl.BlockSpec((1, slabs, 8, lanes), lambda i: (i, 0, 0, 0)),
        out_shape=jax.ShapeDtypeStruct((B, slabs, 8, lanes), jnp.float32),
    )(xf)
    return out.reshape(B, C, H, W)


# speedup vs baseline: 2.4774x; 2.4774x over previous
import jax
import jax.numpy as jnp
from jax.experimental import pallas as pl

GAMMA_K = 0.1
_INT_MIN = -(2 ** 31)


_NCHAIN = 8


def _select_mask_body(x_ref, o_ref, *, kth: int):
    xb = x_ref[0]
    xz = xb + 0.0
    b = jax.lax.bitcast_convert_type(xz, jnp.int32)
    u = jnp.where(b < 0, b ^ jnp.int32(0x7FFFFFFF), b)
    imin = jnp.int32(_INT_MIN)
    rows = u.shape[0]
    step = rows // _NCHAIN

    def it(i, zb):
        bit = jnp.int32(31) - i
        cand = zb | jnp.left_shift(jnp.int32(1), bit)
        z = cand ^ imin
        parts = [
            jnp.sum((u[s * step:(s + 1) * step] >= z).astype(jnp.int32))
            for s in range(_NCHAIN)
        ]
        cnt = sum(parts)
        return jnp.where(cnt >= kth, cand, zb)

    zb = jax.lax.fori_loop(0, 32, it, jnp.int32(0))
    t = zb ^ imin
    o_ref[0] = jnp.where(u < t, xb, 0.0)


def kernel(x):
    B, C, H, W = x.shape
    n = C * H * W
    kth = int(GAMMA_K * n)
    lanes = 1024
    rows = n // lanes
    xf = x.reshape(B, rows, lanes)

    out = pl.pallas_call(
        lambda x_ref, o_ref: _select_mask_body(x_ref, o_ref, kth=kth),
        grid=(B,),
        in_specs=[pl.BlockSpec((1, rows, lanes), lambda i: (i, 0, 0))],
        out_specs=pl.BlockSpec((1, rows, lanes), lambda i: (i, 0, 0)),
        out_shape=jax.ShapeDtypeStruct((B, rows, lanes), jnp.float32),
    )(xf)
    return out.reshape(B, C, H, W)


# speedup vs baseline: 3.0380x; 1.2263x over previous
import jax
import jax.numpy as jnp
from jax.experimental import pallas as pl

GAMMA_K = 0.1
_ROWS_PER_STEP = 2
_NCHAIN = 16


def _float_of_biased(zb):
    s = zb ^ jnp.int32(-2147483648)
    fb = jnp.where(s < 0, s ^ jnp.int32(0x7FFFFFFF), s)
    return jax.lax.bitcast_convert_type(fb, jnp.float32)


def _count_ge(xr, z_f):
    rows = xr.shape[0]
    nchain = _NCHAIN if rows >= _NCHAIN else rows
    step = rows // nchain
    parts = [
        jnp.sum((xr[s * step:(s + 1) * step] >= z_f).astype(jnp.int32))
        for s in range(nchain)
    ]
    return sum(parts)


def _select_mask_body(x_ref, o_ref, *, kth: int):
    xs = [x_ref[r] for r in range(_ROWS_PER_STEP)]

    def it(i, zbs):
        bit = jnp.int32(31) - i
        bitv = jnp.left_shift(jnp.int32(1), bit)
        out = []
        for r in range(_ROWS_PER_STEP):
            cand = zbs[r] | bitv
            cnt = _count_ge(xs[r], _float_of_biased(cand))
            out.append(jnp.where(cnt >= kth, cand, zbs[r]))
        return tuple(out)

    zbs = jax.lax.fori_loop(
        0, 32, it, tuple(jnp.int32(0) for _ in range(_ROWS_PER_STEP)))
    for r in range(_ROWS_PER_STEP):
        t_f = _float_of_biased(zbs[r])
        o_ref[r] = jnp.where(xs[r] < t_f, xs[r], 0.0)


def kernel(x):
    B, C, H, W = x.shape
    n = C * H * W
    kth = int(GAMMA_K * n)
    lanes = 1024
    rows = n // lanes
    xf = x.reshape(B, rows, lanes)
    g = _ROWS_PER_STEP

    out = pl.pallas_call(
        lambda x_ref, o_ref: _select_mask_body(x_ref, o_ref, kth=kth),
        grid=(B // g,),
        in_specs=[pl.BlockSpec((g, rows, lanes), lambda i: (i, 0, 0))],
        out_specs=pl.BlockSpec((g, rows, lanes), lambda i: (i, 0, 0)),
        out_shape=jax.ShapeDtypeStruct((B, rows, lanes), jnp.float32),
    )(xf)
    return out.reshape(B, C, H, W)


# speedup vs baseline: 3.0931x; 1.0181x over previous
import jax
import jax.numpy as jnp
from jax.experimental import pallas as pl

GAMMA_K = 0.1
_ROWS_PER_STEP = 2
_NCHAIN = 16


def _float_of_biased(zb):
    s = zb ^ jnp.int32(-2147483648)
    fb = jnp.where(s < 0, s ^ jnp.int32(0x7FFFFFFF), s)
    return jax.lax.bitcast_convert_type(fb, jnp.float32)


def _count_ge(xr, z_f):
    rows = xr.shape[0]
    nchain = _NCHAIN if rows >= _NCHAIN else rows
    step = rows // nchain
    parts = [
        jnp.sum((xr[s * step:(s + 1) * step] >= z_f).astype(jnp.int32))
        for s in range(nchain)
    ]
    return sum(parts)


def _select_mask_body(x_ref, o_ref, *, kth: int):
    def it(i, zbs):
        bit = jnp.int32(31) - i
        bitv = jnp.left_shift(jnp.int32(1), bit)
        out = []
        for r in range(_ROWS_PER_STEP):
            cand = zbs[r] | bitv
            cnt = _count_ge(x_ref[r], _float_of_biased(cand))
            out.append(jnp.where(cnt >= kth, cand, zbs[r]))
        return tuple(out)

    zbs = jax.lax.fori_loop(
        0, 32, it, tuple(jnp.int32(0) for _ in range(_ROWS_PER_STEP)))
    for r in range(_ROWS_PER_STEP):
        t_f = _float_of_biased(zbs[r])
        xr = x_ref[r]
        o_ref[r] = jnp.where(xr < t_f, xr, 0.0)


def kernel(x):
    B, C, H, W = x.shape
    n = C * H * W
    kth = int(GAMMA_K * n)
    lanes = 1024
    rows = n // lanes
    xf = x.reshape(B, rows, lanes)
    g = _ROWS_PER_STEP

    out = pl.pallas_call(
        lambda x_ref, o_ref: _select_mask_body(x_ref, o_ref, kth=kth),
        grid=(B // g,),
        in_specs=[pl.BlockSpec((g, rows, lanes), lambda i: (i, 0, 0))],
        out_specs=pl.BlockSpec((g, rows, lanes), lambda i: (i, 0, 0)),
        out_shape=jax.ShapeDtypeStruct((B, rows, lanes), jnp.float32),
    )(xf)
    return out.reshape(B, C, H, W)


# speedup vs baseline: 3.1501x; 1.0184x over previous
import jax
import jax.numpy as jnp
from jax.experimental import pallas as pl

GAMMA_K = 0.1
_ROWS_PER_STEP = 4
_NCHAIN = 16


def _float_of_biased(zb):
    s = zb ^ jnp.int32(-2147483648)
    fb = jnp.where(s < 0, s ^ jnp.int32(0x7FFFFFFF), s)
    return jax.lax.bitcast_convert_type(fb, jnp.float32)


def _count_ge(xr, z_f):
    rows = xr.shape[0]
    nchain = _NCHAIN if rows >= _NCHAIN else rows
    step = rows // nchain
    parts = [
        jnp.sum((xr[s * step:(s + 1) * step] >= z_f).astype(jnp.int32))
        for s in range(nchain)
    ]
    return sum(parts)


def _select_mask_body(x_ref, o_ref, *, kth: int):
    def it(i, zbs):
        bit = jnp.int32(31) - i
        bitv = jnp.left_shift(jnp.int32(1), bit)
        out = []
        for r in range(_ROWS_PER_STEP):
            cand = zbs[r] | bitv
            cnt = _count_ge(x_ref[r], _float_of_biased(cand))
            out.append(jnp.where(cnt >= kth, cand, zbs[r]))
        return tuple(out)

    zbs = jax.lax.fori_loop(
        0, 32, it, tuple(jnp.int32(0) for _ in range(_ROWS_PER_STEP)))
    for r in range(_ROWS_PER_STEP):
        t_f = _float_of_biased(zbs[r])
        xr = x_ref[r]
        o_ref[r] = jnp.where(xr < t_f, xr, 0.0)


def kernel(x):
    B, C, H, W = x.shape
    n = C * H * W
    kth = int(GAMMA_K * n)
    lanes = 1024
    rows = n // lanes
    xf = x.reshape(B, rows, lanes)
    g = _ROWS_PER_STEP

    out = pl.pallas_call(
        lambda x_ref, o_ref: _select_mask_body(x_ref, o_ref, kth=kth),
        grid=(B // g,),
        in_specs=[pl.BlockSpec((g, rows, lanes), lambda i: (i, 0, 0))],
        out_specs=pl.BlockSpec((g, rows, lanes), lambda i: (i, 0, 0)),
        out_shape=jax.ShapeDtypeStruct((B, rows, lanes), jnp.float32),
    )(xf)
    return out.reshape(B, C, H, W)
